# Initial kernel scaffold; baseline (speedup 1.0000x reference)
import jax
import jax.numpy as jnp
from jax.experimental import pallas as pl


def kernel(x, edge_index, edge_curvature, edge_attr, W0, b0, We0, W1, b1, We1, att_src, att_dst, gamma, beta):
    raise NotImplementedError("write your pallas kernel here")


# speedup vs baseline: 5.6220x; 5.6220x over previous
import functools

import jax
import jax.numpy as jnp
from jax import lax
from jax.experimental import pallas as pl
from jax.experimental.pallas import tpu as pltpu
from jax.experimental.pallas import tpu_sc as plsc

N = 10000
E = 320000
D = 128
DE = 16

NC = 2
NS = 16
NT = NC * NS
EPT = E // NT
C = 80
NCH = EPT // C
RPT = 624
W_L0 = 144
W_ATT = 160

_f32 = jnp.float32
_i32 = jnp.int32


def _zero_stage(stage, width):
    def body(e, carry):
        for k in range(width // 16):
            stage[e, pl.ds(k * 16, 16)] = jnp.zeros((16,), _f32)
        return carry
    lax.fori_loop(0, C, body, 0)


def _zero_and_barrier(stage, acc, sid, width):
    _zero_stage(stage, width)
    r0 = sid * RPT
    nchunks = (N - (NS - 1) * RPT + C - 1) // C
    cap = jnp.where(sid == NS - 1, N - (NS - 1) * RPT, RPT)

    def body(r, carry):
        @pl.when(r * C < cap)
        def _():
            pltpu.sync_copy(stage.at[pl.ds(0, C)], acc.at[pl.ds(r0 + r * C, C)])
        return carry
    lax.fori_loop(0, nchunks, body, 0)
    plsc.subcore_barrier()


def _copy_out(stage, acc, out, cid, sid):
    plsc.subcore_barrier()
    r0 = sid * RPT
    nchunks = (N - (NS - 1) * RPT + C - 1) // C
    cap = jnp.where(sid == NS - 1, N - (NS - 1) * RPT, RPT)

    def body(r, carry):
        @pl.when(r * C < cap)
        def _():
            pltpu.sync_copy(acc.at[pl.ds(r0 + r * C, C)], stage.at[pl.ds(0, C)])
            pltpu.sync_copy(stage.at[pl.ds(0, C)],
                            out.at[cid, pl.ds(r0 + r * C, C)])
        return carry
    lax.fori_loop(0, nchunks, body, 0)


def _scale_rows(stage, rows_v, ea_v, coef, width):
    lanes = lax.iota(_i32, 16)

    def body(e, carry):
        c = plsc.load_gather(coef, [jnp.full((16,), 0, _i32) + e])
        for k in range(8):
            stage[e, pl.ds(k * 16, 16)] = rows_v[e, pl.ds(k * 16, 16)] * c
        stage[e, pl.ds(128, 16)] = ea_v[e, :] * c
        return carry
    del lanes
    lax.fori_loop(0, C, body, 0)


def _make_sc_l0():
    mesh = plsc.VectorSubcoreMesh(core_axis_name="c", subcore_axis_name="s")

    @functools.partial(
        pl.kernel,
        out_type=jax.ShapeDtypeStruct((NC, N, W_L0), _f32),
        mesh=mesh,
        compiler_params=pltpu.CompilerParams(use_tc_tiling_on_sc=False, needs_layout_passes=False),
        scratch_types=[
            pltpu.VMEM((C,), _i32),
            pltpu.VMEM((C,), _i32),
            pltpu.VMEM((C,), _f32),
            pltpu.VMEM((C, DE), _f32),
            pltpu.VMEM((C, D), _f32),
            pltpu.VMEM((C, W_L0), _f32),
            pltpu.VMEM((C,), _f32),
            pltpu.VMEM_SHARED((N, W_L0), _f32),
            pltpu.SemaphoreType.DMA,
        ],
    )
    def sc_l0(table, src, dst, curv, ea, out,
              src_v, dst_v, curv_v, ea_v, rows_v, stage, coef, acc, sem):
        cid = lax.axis_index("c")
        sid = lax.axis_index("s")
        base = (cid * NS + sid) * EPT

        _zero_and_barrier(stage, acc, sid, W_L0)

        def chunk(i, carry):
            off = base + i * C
            pltpu.sync_copy(src.at[pl.ds(off, C)], src_v)
            pltpu.sync_copy(dst.at[pl.ds(off, C)], dst_v)
            pltpu.sync_copy(curv.at[pl.ds(off, C)], curv_v)
            pltpu.sync_copy(ea.at[pl.ds(off, C)], ea_v)
            pltpu.async_copy(table.at[src_v], rows_v, sem).wait()
            for j in range(C // 16):
                sl = pl.ds(j * 16, 16)
                cv = curv_v[sl]
                coef[sl] = jnp.where(cv > 0.0, 1.0, 0.0).astype(_f32)
            _scale_rows(stage, rows_v, ea_v, coef, W_L0)
            pltpu.sync_copy(stage, acc.at[dst_v], add=True)
            return carry

        lax.fori_loop(0, NCH, chunk, 0)
        _copy_out(stage, acc, out, cid, sid)

    return sc_l0


def _make_sc_att():
    mesh = plsc.VectorSubcoreMesh(core_axis_name="c", subcore_axis_name="s")

    @functools.partial(
        pl.kernel,
        out_type=jax.ShapeDtypeStruct((NC, N, W_ATT), _f32),
        mesh=mesh,
        compiler_params=pltpu.CompilerParams(use_tc_tiling_on_sc=False, needs_layout_passes=False),
        scratch_types=[
            pltpu.VMEM((C,), _f32),
            pltpu.VMEM((C,), _f32),
            pltpu.VMEM((16,), _f32),
            pltpu.VMEM((C,), _i32),
            pltpu.VMEM((C,), _i32),
            pltpu.VMEM((C,), _f32),
            pltpu.VMEM((C, DE), _f32),
            pltpu.VMEM((C, D), _f32),
            pltpu.VMEM((C, W_ATT), _f32),
            pltpu.VMEM((C,), _f32),
            pltpu.VMEM_SHARED((N, W_ATT), _f32),
            pltpu.SemaphoreType.DMA,
        ],
    )
    def sc_att(table, src, dst, curv, ea, a_node, b_node, mvec, out,
               a_v, b_v, mv, src_v, dst_v, curv_v, ea_v, rows_v, stage,
               coef, acc, sem):
        cid = lax.axis_index("c")
        sid = lax.axis_index("s")
        base = (cid * NS + sid) * EPT

        pltpu.sync_copy(mvec.at[0], mv)
        _zero_and_barrier(stage, acc, sid, W_ATT)

        col144 = jnp.full((16,), 144, _i32)

        def chunk(i, carry):
            off = base + i * C
            pltpu.sync_copy(src.at[pl.ds(off, C)], src_v)
            pltpu.sync_copy(dst.at[pl.ds(off, C)], dst_v)
            pltpu.sync_copy(curv.at[pl.ds(off, C)], curv_v)
            pltpu.sync_copy(ea.at[pl.ds(off, C)], ea_v)
            d1 = pltpu.async_copy(table.at[src_v], rows_v, sem)
            d2 = pltpu.async_copy(a_node.at[src_v], a_v, sem)
            d3 = pltpu.async_copy(b_node.at[dst_v], b_v, sem)
            d1.wait()
            d2.wait()
            d3.wait()
            mvv = mv[...]
            for j in range(C // 16):
                sl = pl.ds(j * 16, 16)
                cv = curv_v[sl]
                av = a_v[sl]
                bv = b_v[sl]
                s = av + bv
                s = jnp.where(s >= 0.0, s, s * 0.2)
                ex = jnp.exp(s - mvv)
                gate = jnp.where(cv < 0.0, 1.0, 0.0).astype(_f32)
                coef[sl] = ex * gate
                ridx = lax.iota(_i32, 16) + j * 16
                plsc.store_scatter(stage, [ridx, col144], ex)
            _scale_rows(stage, rows_v, ea_v, coef, W_ATT)
            pltpu.sync_copy(stage, acc.at[dst_v], add=True)
            return carry

        lax.fori_loop(0, NCH, chunk, 0)
        _copy_out(stage, acc, out, cid, sid)

    return sc_att


_sc_l0 = _make_sc_l0()
_sc_att = _make_sc_att()


BLK = 2000
NBLK = N // BLK
_NEG_INF = -3.0e38


def _leaky(v):
    return jnp.where(v >= 0.0, v, v * 0.2)


def _tc_in_proj(x, w, b):
    def body(x_ref, w_ref, b_ref, o_ref):
        o_ref[...] = jnp.dot(x_ref[...], w_ref[...],
                             preferred_element_type=_f32) + b_ref[...]
    return pl.pallas_call(
        body,
        grid=(NBLK,),
        in_specs=[pl.BlockSpec((BLK, D), lambda i: (i, 0)),
                  pl.BlockSpec((D, D), lambda i: (0, 0)),
                  pl.BlockSpec((D,), lambda i: (0,))],
        out_specs=pl.BlockSpec((BLK, D), lambda i: (i, 0)),
        out_shape=jax.ShapeDtypeStruct((N, D), _f32))(x, w, b)


def _tc_combine_stats(parts0, we0):
    def body(p_ref, we_ref, o_ref, s_ref, q_ref):
        @pl.when(pl.program_id(0) == 0)
        def _():
            s_ref[...] = jnp.zeros_like(s_ref)
            q_ref[...] = jnp.zeros_like(q_ref)
        acc = p_ref[0] + p_ref[1]
        out0 = acc[:, :D] + jnp.dot(acc[:, D:], we_ref[...],
                                    preferred_element_type=_f32)
        o_ref[...] = out0
        s_ref[...] += jnp.sum(out0, axis=0, keepdims=True)
        q_ref[...] += jnp.sum(jnp.square(out0), axis=0, keepdims=True)
    return pl.pallas_call(
        body,
        grid=(NBLK,),
        in_specs=[pl.BlockSpec((2, BLK, W_L0), lambda i: (0, i, 0)),
                  pl.BlockSpec((DE, D), lambda i: (0, 0))],
        out_specs=(pl.BlockSpec((BLK, D), lambda i: (i, 0)),
                   pl.BlockSpec((1, D), lambda i: (0, 0)),
                   pl.BlockSpec((1, D), lambda i: (0, 0))),
        out_shape=(jax.ShapeDtypeStruct((N, D), _f32),
                   jax.ShapeDtypeStruct((1, D), _f32),
                   jax.ShapeDtypeStruct((1, D), _f32)),
    )(parts0, we0)


def _tc_bn_proj(out0, sums, sumsq, gamma, beta, w1, b1, asrc, adst):
    def body(o_ref, s_ref, q_ref, g_ref, bt_ref, w1_ref, b1_ref, as_ref,
             ad_ref, t1_ref, a_ref, b_ref, ma_ref, mb_ref, m_ref):
        @pl.when(pl.program_id(0) == 0)
        def _():
            ma_ref[...] = jnp.full_like(ma_ref, _NEG_INF)
            mb_ref[...] = jnp.full_like(mb_ref, _NEG_INF)
        mean = s_ref[...] / N
        var = q_ref[...] / N - jnp.square(mean)
        h = (o_ref[...] - mean) / jnp.sqrt(var + 1e-5) * g_ref[...] + bt_ref[...]
        h = jnp.maximum(h, 0.0)
        t1 = jnp.dot(h, w1_ref[...], preferred_element_type=_f32) + b1_ref[...]
        t1_ref[...] = t1
        a = jnp.dot(t1, as_ref[...], preferred_element_type=_f32)
        b = jnp.dot(t1, ad_ref[...], preferred_element_type=_f32)
        a_ref[...] = a
        b_ref[...] = b
        ma_ref[...] = jnp.maximum(ma_ref[...], jnp.max(a))
        mb_ref[...] = jnp.maximum(mb_ref[...], jnp.max(b))
        m_ref[...] = jnp.broadcast_to(_leaky(ma_ref[0, 0] + mb_ref[0, 0]),
                                      (1, 16))
    return pl.pallas_call(
        body,
        grid=(NBLK,),
        in_specs=[pl.BlockSpec((BLK, D), lambda i: (i, 0)),
                  pl.BlockSpec((1, D), lambda i: (0, 0)),
                  pl.BlockSpec((1, D), lambda i: (0, 0)),
                  pl.BlockSpec((D,), lambda i: (0,)),
                  pl.BlockSpec((D,), lambda i: (0,)),
                  pl.BlockSpec((D, D), lambda i: (0, 0)),
                  pl.BlockSpec((D,), lambda i: (0,)),
                  pl.BlockSpec((D, 1), lambda i: (0, 0)),
                  pl.BlockSpec((D, 1), lambda i: (0, 0))],
        out_specs=(pl.BlockSpec((BLK, D), lambda i: (i, 0)),
                   pl.BlockSpec((BLK, 1), lambda i: (i, 0)),
                   pl.BlockSpec((BLK, 1), lambda i: (i, 0)),
                   pl.BlockSpec((1, 1), lambda i: (0, 0)),
                   pl.BlockSpec((1, 1), lambda i: (0, 0)),
                   pl.BlockSpec((1, 16), lambda i: (0, 0))),
        out_shape=(jax.ShapeDtypeStruct((N, D), _f32),
                   jax.ShapeDtypeStruct((N, 1), _f32),
                   jax.ShapeDtypeStruct((N, 1), _f32),
                   jax.ShapeDtypeStruct((1, 1), _f32),
                   jax.ShapeDtypeStruct((1, 1), _f32),
                   jax.ShapeDtypeStruct((1, 16), _f32)),
    )(out0, sums, sumsq, gamma, beta, w1, b1, asrc, adst)


def _tc_hop_combine(parts, we1, asrc, adst, with_att):
    def body_att(p_ref, we_ref, as_ref, ad_ref, t_ref, a_ref, b_ref,
                 ma_ref, mb_ref, m_ref):
        @pl.when(pl.program_id(0) == 0)
        def _():
            ma_ref[...] = jnp.full_like(ma_ref, _NEG_INF)
            mb_ref[...] = jnp.full_like(mb_ref, _NEG_INF)
        acc = p_ref[0] + p_ref[1]
        den = acc[:, 144:145] + 1e-16
        t = (acc[:, :D] + jnp.dot(acc[:, D:W_L0], we_ref[...],
                                  preferred_element_type=_f32)) / den
        t_ref[...] = t
        a = jnp.dot(t, as_ref[...], preferred_element_type=_f32)
        b = jnp.dot(t, ad_ref[...], preferred_element_type=_f32)
        a_ref[...] = a
        b_ref[...] = b
        ma_ref[...] = jnp.maximum(ma_ref[...], jnp.max(a))
        mb_ref[...] = jnp.maximum(mb_ref[...], jnp.max(b))
        m_ref[...] = jnp.broadcast_to(_leaky(ma_ref[0, 0] + mb_ref[0, 0]),
                                      (1, 16))

    def body_plain(p_ref, we_ref, t_ref):
        acc = p_ref[0] + p_ref[1]
        den = acc[:, 144:145] + 1e-16
        t_ref[...] = (acc[:, :D] + jnp.dot(acc[:, D:W_L0], we_ref[...],
                                           preferred_element_type=_f32)) / den

    if with_att:
        return pl.pallas_call(
            body_att,
            grid=(NBLK,),
            in_specs=[pl.BlockSpec((2, BLK, W_ATT), lambda i: (0, i, 0)),
                      pl.BlockSpec((DE, D), lambda i: (0, 0)),
                      pl.BlockSpec((D, 1), lambda i: (0, 0)),
                      pl.BlockSpec((D, 1), lambda i: (0, 0))],
            out_specs=(pl.BlockSpec((BLK, D), lambda i: (i, 0)),
                       pl.BlockSpec((BLK, 1), lambda i: (i, 0)),
                       pl.BlockSpec((BLK, 1), lambda i: (i, 0)),
                       pl.BlockSpec((1, 1), lambda i: (0, 0)),
                       pl.BlockSpec((1, 1), lambda i: (0, 0)),
                       pl.BlockSpec((1, 16), lambda i: (0, 0))),
            out_shape=(jax.ShapeDtypeStruct((N, D), _f32),
                       jax.ShapeDtypeStruct((N, 1), _f32),
                       jax.ShapeDtypeStruct((N, 1), _f32),
                       jax.ShapeDtypeStruct((1, 1), _f32),
                       jax.ShapeDtypeStruct((1, 1), _f32),
                       jax.ShapeDtypeStruct((1, 16), _f32)),
        )(parts, we1, asrc, adst)
    return pl.pallas_call(
        body_plain,
        grid=(NBLK,),
        in_specs=[pl.BlockSpec((2, BLK, W_ATT), lambda i: (0, i, 0)),
                  pl.BlockSpec((DE, D), lambda i: (0, 0))],
        out_specs=pl.BlockSpec((BLK, D), lambda i: (i, 0)),
        out_shape=jax.ShapeDtypeStruct((N, D), _f32))(parts, we1)


def kernel(x, edge_index, edge_curvature, edge_attr, W0, b0, We0, W1, b1,
           We1, att_src, att_dst, gamma, beta):
    src = edge_index[0]
    dst = edge_index[1]
    asrc = att_src.reshape(D, 1)
    adst = att_dst.reshape(D, 1)

    t0 = _tc_in_proj(x, W0, b0)
    parts0 = _sc_l0(t0, src, dst, edge_curvature, edge_attr)
    out0, sums, sumsq = _tc_combine_stats(parts0, We0)
    t1, a1, b1n, _, _, m1 = _tc_bn_proj(out0, sums, sumsq, gamma, beta,
                                        W1, b1, asrc, adst)
    parts1 = _sc_att(t1, src, dst, edge_curvature, edge_attr,
                     a1.reshape(N), b1n.reshape(N), m1)
    t2, a2, b2n, _, _, m2 = _tc_hop_combine(parts1, We1, asrc, adst, True)
    parts2 = _sc_att(t2, src, dst, edge_curvature, edge_attr,
                     a2.reshape(N), b2n.reshape(N), m2)
    return _tc_hop_combine(parts2, We1, None, None, False)
